# baseline (speedup 1.0000x reference)
import functools

import jax
import jax.numpy as jnp
from jax.experimental import pallas as pl


def _loss_kernel(an_ref, gtb_ref, gtl_ref, pl_ref, pc_ref,
                 ll_ref, lc_ref, np_ref, *, G, C, A, SA, LA):
    f32 = jnp.float32
    ax = an_ref[0]
    ay = an_ref[1]
    aw = an_ref[2]
    ah = an_ref[3]
    ax1 = ax - aw * 0.5
    ay1 = ay - ah * 0.5
    ax2 = ax + aw * 0.5
    ay2 = ay + ah * 0.5
    area_a = (ax2 - ax1) * (ay2 - ay1)

    sl_iota = jax.lax.broadcasted_iota(jnp.int32, (SA, LA), 0)
    ln_iota = jax.lax.broadcasted_iota(jnp.int32, (SA, LA), 1)
    flat = sl_iota * LA + ln_iota

    best = jnp.full((SA, LA), -1.0, f32)
    bidx = jnp.zeros((SA, LA), jnp.int32)
    bps = []
    gbox = []
    for g in range(G):
        bx1 = gtb_ref[0, g, 0]
        by1 = gtb_ref[0, g, 1]
        bx2 = gtb_ref[0, g, 2]
        by2 = gtb_ref[0, g, 3]
        gbox.append((bx1, by1, bx2, by2))
        area_b = (bx2 - bx1) * (by2 - by1)
        iw = jnp.maximum(jnp.minimum(bx2, ax2) - jnp.maximum(bx1, ax1), 0.0)
        ih = jnp.maximum(jnp.minimum(by2, ay2) - jnp.maximum(by1, ay1), 0.0)
        inter = iw * ih
        iou = inter / (area_b + area_a - inter)
        upd = iou > best
        best = jnp.where(upd, iou, best)
        bidx = jnp.where(upd, g, bidx)
        mx = jnp.max(iou)
        bps.append(jnp.min(jnp.where(iou == mx, flat, A)))

    fmask = jnp.zeros((SA, LA), jnp.bool_)
    fidx = jnp.zeros((SA, LA), jnp.int32)
    for g in range(G):
        fm = flat == bps[g]
        fmask = jnp.logical_or(fmask, fm)
        fidx = jnp.where(fm, g, fidx)
    bto = jnp.where(fmask, 2.0, best)
    bti = jnp.where(fmask, fidx, bidx)

    labv = jnp.zeros((SA, LA), jnp.int32)
    mx1 = jnp.zeros((SA, LA), f32)
    my1 = jnp.zeros((SA, LA), f32)
    mx2 = jnp.zeros((SA, LA), f32)
    my2 = jnp.zeros((SA, LA), f32)
    for g in range(G):
        m = bti == g
        labv = jnp.where(m, gtl_ref[0, 0, g], labv)
        bx1, by1, bx2, by2 = gbox[g]
        mx1 = jnp.where(m, bx1, mx1)
        my1 = jnp.where(m, by1, my1)
        mx2 = jnp.where(m, bx2, mx2)
        my2 = jnp.where(m, by2, my2)

    conf = jnp.where(bto < 0.5, 0, labv)
    pos = conf > 0
    posf = pos.astype(f32)
    npos_i = jnp.sum(pos.astype(jnp.int32))

    mcx = (mx1 + mx2) * 0.5
    mcy = (my1 + my2) * 0.5
    mw = mx2 - mx1
    mh = my2 - my1
    tcx = (mcx - ax) / (0.1 * aw)
    tcy = (mcy - ay) / (0.1 * ah)
    tw = jnp.log(jnp.maximum(mw / aw, 1e-8)) * 5.0
    th = jnp.log(jnp.maximum(mh / ah, 1e-8)) * 5.0
    ll = jnp.float32(0.0)
    for c, t in enumerate((tcx, tcy, tw, th)):
        d = pl_ref[0, c] - t
        ad = jnp.abs(d)
        sl1 = jnp.where(ad < 1.0, 0.5 * d * d, ad - 0.5)
        ll = ll + jnp.sum(sl1 * posf)

    es = jnp.zeros((SA, LA), f32)
    tgt = jnp.zeros((SA, LA), f32)
    for c in range(C):
        xc = pc_ref[0, c]
        es = es + jnp.exp(xc)
        tgt = jnp.where(conf == c, xc, tgt)
    ce = jnp.log(es) - tgt
    ce_pos_sum = jnp.sum(jnp.where(pos, ce, 0.0))
    temp = jnp.where(pos, 0.0, jnp.maximum(ce, 0.0))

    k = jnp.minimum(3 * npos_i, A - 1)
    ti = jax.lax.bitcast_convert_type(temp, jnp.int32)

    def body(_, lh):
        lo, hi = lh
        mid = lo + (hi - lo) // 2
        cnt = jnp.sum((ti > mid).astype(jnp.int32))
        le = cnt <= (k - 1)
        return (jnp.where(le, lo, mid), jnp.where(le, mid, hi))

    lo0 = jnp.int32(-1)
    hi0 = jnp.int32(0x7F800000)
    _, vk_bits = jax.lax.fori_loop(0, 31, body, (lo0, hi0))
    gt_mask = ti > vk_bits
    cnt_gt = jnp.sum(gt_mask.astype(jnp.int32))
    sum_gt = jnp.sum(jnp.where(gt_mask, temp, 0.0))
    vk = jnp.max(jnp.where(gt_mask, 0.0, temp))
    topk = sum_gt + (k - cnt_gt).astype(f32) * vk
    lc = ce_pos_sum + topk

    ll_ref[...] = jnp.full((1, 1, 128), ll, f32)
    lc_ref[...] = jnp.full((1, 1, 128), lc, f32)
    np_ref[...] = jnp.full((1, 1, 128), npos_i.astype(f32), f32)


def kernel(p_locs, p_conf, gt_bboxes, gt_labels, anchors):
    B, A, C = p_conf.shape
    G = gt_bboxes.shape[1]
    SA = 32
    LA = A // SA
    an_r = anchors.T.reshape(4, SA, LA)
    pl_r = p_locs.transpose(0, 2, 1).reshape(B, 4, SA, LA)
    eye_c = jnp.eye(C, dtype=p_conf.dtype)
    pc_t = jnp.einsum('bac,dc->bda', p_conf, eye_c)
    pc_r = pc_t.reshape(B, C, SA, LA)
    gtl_r = gt_labels.astype(jnp.int32).reshape(B, 1, G)

    out_shape = [jax.ShapeDtypeStruct((B, 1, 128), jnp.float32)] * 3
    ll, lc, npf = pl.pallas_call(
        functools.partial(_loss_kernel, G=G, C=C, A=A, SA=SA, LA=LA),
        grid=(B,),
        in_specs=[
            pl.BlockSpec((4, SA, LA), lambda b: (0, 0, 0)),
            pl.BlockSpec((1, G, 4), lambda b: (b, 0, 0)),
            pl.BlockSpec((1, 1, G), lambda b: (b, 0, 0)),
            pl.BlockSpec((1, 4, SA, LA), lambda b: (b, 0, 0, 0)),
            pl.BlockSpec((1, C, SA, LA), lambda b: (b, 0, 0, 0)),
        ],
        out_specs=[pl.BlockSpec((1, 1, 128), lambda b: (b, 0, 0))] * 3,
        out_shape=out_shape,
    )(an_r, gt_bboxes, gtl_r, pl_r, pc_r)

    ll_s = jnp.sum(ll[:, 0, 0])
    lc_s = jnp.sum(lc[:, 0, 0])
    np_s = jnp.sum(npf[:, 0, 0])
    n = jnp.maximum(np_s, 1.0)
    return (ll_s / n, lc_s / n)


# speedup vs baseline: 1.2803x; 1.2803x over previous
import functools

import jax
import jax.numpy as jnp
from jax.experimental import pallas as pl
from jax.experimental.pallas import tpu as pltpu


def _sum23(x):
    return jnp.sum(jnp.sum(x, axis=2, keepdims=True), axis=1, keepdims=True)


def _max23(x):
    return jnp.max(jnp.max(x, axis=2, keepdims=True), axis=1, keepdims=True)


def _loss_kernel(an_ref, gtb_ref, gtl_ref, pl_ref, pc_ref,
                 ll_ref, lc_ref,
                 temp_scr, ll_scr, lc_scr, np_scr,
                 *, B, G, C, A, SA, LA):
    f32 = jnp.float32
    b = pl.program_id(0)
    ax = an_ref[0]
    ay = an_ref[1]
    aw = an_ref[2]
    ah = an_ref[3]
    ax1 = ax - aw * 0.5
    ay1 = ay - ah * 0.5
    ax2 = ax + aw * 0.5
    ay2 = ay + ah * 0.5
    area_a = (ax2 - ax1) * (ay2 - ay1)

    sl_iota = jax.lax.broadcasted_iota(jnp.int32, (SA, LA), 0)
    ln_iota = jax.lax.broadcasted_iota(jnp.int32, (SA, LA), 1)
    flat = sl_iota * LA + ln_iota

    best = jnp.full((SA, LA), -1.0, f32)
    bidx = jnp.zeros((SA, LA), jnp.int32)
    bps = []
    gbox = []
    for g in range(G):
        bx1 = gtb_ref[0, g, 0]
        by1 = gtb_ref[0, g, 1]
        bx2 = gtb_ref[0, g, 2]
        by2 = gtb_ref[0, g, 3]
        gbox.append((bx1, by1, bx2, by2))
        area_b = (bx2 - bx1) * (by2 - by1)
        iw = jnp.maximum(jnp.minimum(bx2, ax2) - jnp.maximum(bx1, ax1), 0.0)
        ih = jnp.maximum(jnp.minimum(by2, ay2) - jnp.maximum(by1, ay1), 0.0)
        inter = iw * ih
        iou = inter / (area_b + area_a - inter)
        upd = iou > best
        best = jnp.where(upd, iou, best)
        bidx = jnp.where(upd, g, bidx)
        mx = jnp.max(iou)
        bps.append(jnp.min(jnp.where(iou == mx, flat, A)))

    fmask = jnp.zeros((SA, LA), jnp.bool_)
    fidx = jnp.zeros((SA, LA), jnp.int32)
    for g in range(G):
        fm = flat == bps[g]
        fmask = jnp.logical_or(fmask, fm)
        fidx = jnp.where(fm, g, fidx)
    bto = jnp.where(fmask, 2.0, best)
    bti = jnp.where(fmask, fidx, bidx)

    labv = jnp.zeros((SA, LA), jnp.int32)
    mx1 = jnp.zeros((SA, LA), f32)
    my1 = jnp.zeros((SA, LA), f32)
    mx2 = jnp.zeros((SA, LA), f32)
    my2 = jnp.zeros((SA, LA), f32)
    for g in range(G):
        m = bti == g
        labv = jnp.where(m, gtl_ref[0, 0, g], labv)
        bx1, by1, bx2, by2 = gbox[g]
        mx1 = jnp.where(m, bx1, mx1)
        my1 = jnp.where(m, by1, my1)
        mx2 = jnp.where(m, bx2, mx2)
        my2 = jnp.where(m, by2, my2)

    conf = jnp.where(bto < 0.5, 0, labv)
    pos = conf > 0
    posf = pos.astype(f32)
    npos = jnp.sum(posf)

    mcx = (mx1 + mx2) * 0.5
    mcy = (my1 + my2) * 0.5
    mw = mx2 - mx1
    mh = my2 - my1
    tcx = (mcx - ax) / (0.1 * aw)
    tcy = (mcy - ay) / (0.1 * ah)
    tw = jnp.log(jnp.maximum(mw / aw, 1e-8)) * 5.0
    th = jnp.log(jnp.maximum(mh / ah, 1e-8)) * 5.0
    ll = jnp.float32(0.0)
    for c, t in enumerate((tcx, tcy, tw, th)):
        d = pl_ref[0, c] - t
        ad = jnp.abs(d)
        sl1 = jnp.where(ad < 1.0, 0.5 * d * d, ad - 0.5)
        ll = ll + jnp.sum(sl1 * posf)

    es = jnp.zeros((SA, LA), f32)
    tgt = jnp.zeros((SA, LA), f32)
    for c in range(C):
        xc = pc_ref[0, c]
        es = es + jnp.exp(xc)
        tgt = jnp.where(conf == c, xc, tgt)
    ce = jnp.log(es) - tgt
    ce_pos_sum = jnp.sum(jnp.where(pos, ce, 0.0))

    temp_scr[b] = jnp.where(pos, 0.0, jnp.maximum(ce, 0.0))
    ll_scr[pl.ds(b, 1), :] = jnp.full((1, 128), ll, f32)
    lc_scr[pl.ds(b, 1), :] = jnp.full((1, 128), ce_pos_sum, f32)
    np_scr[pl.ds(b, 1), :] = jnp.full((1, 128), npos, f32)

    @pl.when(b == B - 1)
    def _finalize():
        t3 = temp_scr[...]
        ti3 = jax.lax.bitcast_convert_type(t3, jnp.int32)
        npv = np_scr[:, 0:1].reshape(B, 1, 1)
        k3 = jnp.minimum(3 * npv.astype(jnp.int32), A - 1)

        def body(_, lh):
            lo, hi = lh
            mid = lo + (hi - lo) // 2
            cnt = _sum23((ti3 > mid).astype(jnp.int32))
            le = cnt <= (k3 - 1)
            return (jnp.where(le, lo, mid), jnp.where(le, mid, hi))

        lo0 = jnp.full((B, 1, 1), -1, jnp.int32)
        hi0 = jnp.full((B, 1, 1), 0x7F800000, jnp.int32)
        _, vkb = jax.lax.fori_loop(0, 31, body, (lo0, hi0))
        gtm = ti3 > vkb
        cnt_gt = _sum23(gtm.astype(jnp.int32))
        sum_gt = _sum23(jnp.where(gtm, t3, 0.0))
        vk = _max23(jnp.where(gtm, 0.0, t3))
        topk = sum_gt + (k3 - cnt_gt).astype(f32) * vk

        ll_tot = jnp.sum(ll_scr[:, 0:1])
        lc_tot = jnp.sum(lc_scr[:, 0:1]) + jnp.sum(topk)
        np_tot = jnp.sum(np_scr[:, 0:1])
        n = jnp.maximum(np_tot, 1.0)
        ll_ref[...] = jnp.full((1, 1, 128), ll_tot / n, f32)
        lc_ref[...] = jnp.full((1, 1, 128), lc_tot / n, f32)


def kernel(p_locs, p_conf, gt_bboxes, gt_labels, anchors):
    B, A, C = p_conf.shape
    G = gt_bboxes.shape[1]
    SA = 32
    LA = A // SA
    an_r = anchors.T.reshape(4, SA, LA)
    pl_r = p_locs.transpose(0, 2, 1).reshape(B, 4, SA, LA)
    pc_r = p_conf.transpose(0, 2, 1).reshape(B, C, SA, LA)
    gtl_r = gt_labels.astype(jnp.int32).reshape(B, 1, G)

    out_shape = [jax.ShapeDtypeStruct((1, 1, 128), jnp.float32)] * 2
    ll, lc = pl.pallas_call(
        functools.partial(_loss_kernel, B=B, G=G, C=C, A=A, SA=SA, LA=LA),
        grid=(B,),
        in_specs=[
            pl.BlockSpec((4, SA, LA), lambda b: (0, 0, 0)),
            pl.BlockSpec((1, G, 4), lambda b: (b, 0, 0)),
            pl.BlockSpec((1, 1, G), lambda b: (b, 0, 0)),
            pl.BlockSpec((1, 4, SA, LA), lambda b: (b, 0, 0, 0)),
            pl.BlockSpec((1, C, SA, LA), lambda b: (b, 0, 0, 0)),
        ],
        out_specs=[pl.BlockSpec((1, 1, 128), lambda b: (0, 0, 0))] * 2,
        out_shape=out_shape,
        scratch_shapes=[
            pltpu.VMEM((B, SA, LA), jnp.float32),
            pltpu.VMEM((B, 128), jnp.float32),
            pltpu.VMEM((B, 128), jnp.float32),
            pltpu.VMEM((B, 128), jnp.float32),
        ],
    )(an_r, gt_bboxes, gtl_r, pl_r, pc_r)

    return (ll[0, 0, 0], lc[0, 0, 0])


# speedup vs baseline: 1.8299x; 1.4293x over previous
import functools

import jax
import jax.numpy as jnp
from jax.experimental import pallas as pl
from jax.experimental.pallas import tpu as pltpu


def _sum23(x):
    return jnp.sum(jnp.sum(x, axis=2, keepdims=True), axis=1, keepdims=True)


def _max23(x):
    return jnp.max(jnp.max(x, axis=2, keepdims=True), axis=1, keepdims=True)


def _loss_kernel(an_ref, gtb_ref, gtl_ref, pl_ref, pc_ref,
                 ll_ref, lc_ref,
                 temp_scr, ll_scr, lc_scr, np_scr,
                 *, B, G, C, A, SA, LA):
    f32 = jnp.float32
    b = pl.program_id(0)
    ax = an_ref[0]
    ay = an_ref[1]
    aw = an_ref[2]
    ah = an_ref[3]
    ax1 = ax - aw * 0.5
    ay1 = ay - ah * 0.5
    ax2 = ax + aw * 0.5
    ay2 = ay + ah * 0.5
    area_a = (ax2 - ax1) * (ay2 - ay1)

    sl_iota = jax.lax.broadcasted_iota(jnp.int32, (SA, LA), 0)
    ln_iota = jax.lax.broadcasted_iota(jnp.int32, (SA, LA), 1)
    flat = sl_iota * LA + ln_iota

    ious = []
    gbox = []
    for g in range(G):
        bx1 = gtb_ref[0, g, 0]
        by1 = gtb_ref[0, g, 1]
        bx2 = gtb_ref[0, g, 2]
        by2 = gtb_ref[0, g, 3]
        gbox.append((bx1, by1, bx2, by2))
        area_b = (bx2 - bx1) * (by2 - by1)
        iw = jnp.maximum(jnp.minimum(bx2, ax2) - jnp.maximum(bx1, ax1), 0.0)
        ih = jnp.maximum(jnp.minimum(by2, ay2) - jnp.maximum(by1, ay1), 0.0)
        inter = iw * ih
        ious.append(inter / (area_b + area_a - inter))
    iou3 = jnp.stack(ious, axis=0)
    g3 = jax.lax.broadcasted_iota(jnp.int32, (G, SA, LA), 0)
    best = jnp.max(iou3, axis=0)
    bidx = jnp.min(jnp.where(iou3 == best[None], g3, G), axis=0)
    rmax = _max23(iou3)
    bp3 = jnp.min(jnp.min(jnp.where(iou3 == rmax, flat[None], A),
                          axis=2, keepdims=True), axis=1, keepdims=True)
    fm3 = flat[None] == bp3
    fidx = jnp.max(jnp.where(fm3, g3, -1), axis=0)
    fmask = fidx >= 0
    bto = jnp.where(fmask, 2.0, best)
    bti = jnp.where(fmask, fidx, bidx)

    labv = jnp.zeros((SA, LA), jnp.int32)
    mx1 = jnp.zeros((SA, LA), f32)
    my1 = jnp.zeros((SA, LA), f32)
    mx2 = jnp.zeros((SA, LA), f32)
    my2 = jnp.zeros((SA, LA), f32)
    for g in range(G):
        m = bti == g
        labv = jnp.where(m, gtl_ref[0, 0, g], labv)
        bx1, by1, bx2, by2 = gbox[g]
        mx1 = jnp.where(m, bx1, mx1)
        my1 = jnp.where(m, by1, my1)
        mx2 = jnp.where(m, bx2, mx2)
        my2 = jnp.where(m, by2, my2)

    conf = jnp.where(bto < 0.5, 0, labv)
    pos = conf > 0
    posf = pos.astype(f32)
    npos = jnp.sum(posf)

    mcx = (mx1 + mx2) * 0.5
    mcy = (my1 + my2) * 0.5
    mw = mx2 - mx1
    mh = my2 - my1
    tcx = (mcx - ax) / (0.1 * aw)
    tcy = (mcy - ay) / (0.1 * ah)
    tw = jnp.log(jnp.maximum(mw / aw, 1e-8)) * 5.0
    th = jnp.log(jnp.maximum(mh / ah, 1e-8)) * 5.0
    ll = jnp.float32(0.0)
    for c, t in enumerate((tcx, tcy, tw, th)):
        d = pl_ref[0, c] - t
        ad = jnp.abs(d)
        sl1 = jnp.where(ad < 1.0, 0.5 * d * d, ad - 0.5)
        ll = ll + jnp.sum(sl1 * posf)

    es = jnp.zeros((SA, LA), f32)
    tgt = jnp.zeros((SA, LA), f32)
    for c in range(C):
        xc = pc_ref[0, c]
        es = es + jnp.exp(xc)
        tgt = jnp.where(conf == c, xc, tgt)
    ce = jnp.log(es) - tgt
    ce_pos_sum = jnp.sum(jnp.where(pos, ce, 0.0))

    temp_scr[b] = jnp.where(pos, 0.0, jnp.maximum(ce, 0.0))
    ll_scr[pl.ds(b, 1), :] = jnp.full((1, 128), ll, f32)
    lc_scr[pl.ds(b, 1), :] = jnp.full((1, 128), ce_pos_sum, f32)
    np_scr[pl.ds(b, 1), :] = jnp.full((1, 128), npos, f32)

    @pl.when(b == B - 1)
    def _finalize():
        t3 = temp_scr[...]
        ti3 = jax.lax.bitcast_convert_type(t3, jnp.int32)
        npv = np_scr[:, 0:1].reshape(B, 1, 1)
        k3 = jnp.minimum(3 * npv.astype(jnp.int32), A - 1)

        def body(_, lh):
            lo, hi = lh
            mid = lo + (hi - lo) // 2
            cnt = _sum23((ti3 > mid).astype(jnp.int32))
            le = cnt <= (k3 - 1)
            return (jnp.where(le, lo, mid), jnp.where(le, mid, hi))

        lo0 = jnp.full((B, 1, 1), -1, jnp.int32)
        hi0 = jnp.full((B, 1, 1), 0x7F800000, jnp.int32)
        _, vkb = jax.lax.fori_loop(0, 31, body, (lo0, hi0))
        gtm = ti3 > vkb
        cnt_gt = _sum23(gtm.astype(jnp.int32))
        sum_gt = _sum23(jnp.where(gtm, t3, 0.0))
        vk = _max23(jnp.where(gtm, 0.0, t3))
        topk = sum_gt + (k3 - cnt_gt).astype(f32) * vk

        ll_tot = jnp.sum(ll_scr[:, 0:1])
        lc_tot = jnp.sum(lc_scr[:, 0:1]) + jnp.sum(topk)
        np_tot = jnp.sum(np_scr[:, 0:1])
        n = jnp.maximum(np_tot, 1.0)
        ll_ref[...] = jnp.full((1, 1, 128), ll_tot / n, f32)
        lc_ref[...] = jnp.full((1, 1, 128), lc_tot / n, f32)


def kernel(p_locs, p_conf, gt_bboxes, gt_labels, anchors):
    B, A, C = p_conf.shape
    G = gt_bboxes.shape[1]
    SA = 32
    LA = A // SA
    an_r = anchors.T.reshape(4, SA, LA)
    pl_r = p_locs.transpose(0, 2, 1).reshape(B, 4, SA, LA)
    pc_r = p_conf.transpose(0, 2, 1).reshape(B, C, SA, LA)
    gtl_r = gt_labels.astype(jnp.int32).reshape(B, 1, G)

    out_shape = [jax.ShapeDtypeStruct((1, 1, 128), jnp.float32)] * 2
    ll, lc = pl.pallas_call(
        functools.partial(_loss_kernel, B=B, G=G, C=C, A=A, SA=SA, LA=LA),
        grid=(B,),
        in_specs=[
            pl.BlockSpec((4, SA, LA), lambda b: (0, 0, 0)),
            pl.BlockSpec((1, G, 4), lambda b: (b, 0, 0)),
            pl.BlockSpec((1, 1, G), lambda b: (b, 0, 0)),
            pl.BlockSpec((1, 4, SA, LA), lambda b: (b, 0, 0, 0)),
            pl.BlockSpec((1, C, SA, LA), lambda b: (b, 0, 0, 0)),
        ],
        out_specs=[pl.BlockSpec((1, 1, 128), lambda b: (0, 0, 0))] * 2,
        out_shape=out_shape,
        scratch_shapes=[
            pltpu.VMEM((B, SA, LA), jnp.float32),
            pltpu.VMEM((B, 128), jnp.float32),
            pltpu.VMEM((B, 128), jnp.float32),
            pltpu.VMEM((B, 128), jnp.float32),
        ],
    )(an_r, gt_bboxes, gtl_r, pl_r, pc_r)

    return (ll[0, 0, 0], lc[0, 0, 0])
